# baseline (speedup 1.0000x reference)
import jax
import jax.numpy as jnp
from jax.experimental import pallas as pl
from jax.experimental.pallas import tpu as pltpu
from jax.experimental.pallas import tpu_sc as plsc

_ROWS = 4
_LANES = 16


def kernel(x, pos_table):
    B, T, D = x.shape
    nb = T // _ROWS
    mesh = plsc.VectorSubcoreMesh(core_axis_name="c", subcore_axis_name="s")

    @pl.kernel(out_type=jax.ShapeDtypeStruct(x.shape, x.dtype), mesh=mesh)
    def sc_kernel(x_hbm, pos_hbm, o_hbm):
        def body(xv, pos, ov):
            for b in range(B):
                xb = xv.at[b]
                ob = ov.at[b]

                @pl.loop(0, _ROWS)
                def _(r, xb=xb, ob=ob):
                    @plsc.parallel_loop(0, D, step=_LANES, unroll=16)
                    def _(c):
                        slc = (pl.ds(r, 1), pl.ds(c, _LANES))
                        ob.at[*slc][...] = xb.at[*slc][...] + pos.at[*slc][...]

        spec3 = pl.BlockSpec((B, _ROWS, D), lambda i: (0, i, 0))
        pltpu.emit_pipeline(
            body,
            grid=(nb,),
            in_specs=[spec3, pl.BlockSpec((_ROWS, D), lambda i: (i, 0))],
            out_specs=[spec3],
            core_axis_name=("c", "s"),
            dimension_semantics=(pltpu.PARALLEL,),
        )(x_hbm, pos_hbm, o_hbm)

    return sc_kernel(x, pos_table)


# speedup vs baseline: 1.0001x; 1.0001x over previous
import jax
import jax.numpy as jnp
from jax.experimental import pallas as pl
from jax.experimental.pallas import tpu as pltpu
from jax.experimental.pallas import tpu_sc as plsc

_ROWS = 4
_LANES = 16


def kernel(x, pos_table):
    B, T, D = x.shape
    nb = T // _ROWS
    mesh = plsc.VectorSubcoreMesh(core_axis_name="c", subcore_axis_name="s")

    @pl.kernel(out_type=jax.ShapeDtypeStruct(x.shape, x.dtype), mesh=mesh)
    def sc_kernel(x_hbm, pos_hbm, o_hbm):
        def body(xv, pos, ov):
            for b in range(B):
                xb = xv.at[b]
                ob = ov.at[b]

                @pl.loop(0, _ROWS)
                def _(r, xb=xb, ob=ob):
                    @plsc.parallel_loop(0, D, step=_LANES, unroll=8)
                    def _(c):
                        slc = (pl.ds(r, 1), pl.ds(c, _LANES))
                        ob.at[*slc][...] = xb.at[*slc][...] + pos.at[*slc][...]

        spec3 = pl.BlockSpec((B, _ROWS, D), lambda i: (0, i, 0))
        pltpu.emit_pipeline(
            body,
            grid=(nb,),
            in_specs=[spec3, pl.BlockSpec((_ROWS, D), lambda i: (i, 0))],
            out_specs=[spec3],
            core_axis_name=("c", "s"),
            dimension_semantics=(pltpu.PARALLEL,),
        )(x_hbm, pos_hbm, o_hbm)

    return sc_kernel(x, pos_table)
